# Initial kernel scaffold; baseline (speedup 1.0000x reference)
import jax
import jax.numpy as jnp
from jax.experimental import pallas as pl


def kernel(facts, fact_logits, aggregator_logits, rule_strength_raw, proj_W, ln_gamma, ln_beta):
    raise NotImplementedError("write your pallas kernel here")


# speedup vs baseline: 21.2194x; 21.2194x over previous
import jax
import jax.numpy as jnp
from jax.experimental import pallas as pl

B, D, R = 1024, 128, 256
K_FACTS, K_RULES = 2, 8


def _rule_layer_body(facts_ref, flt_ref, aggT_ref, rs_ref, projWT_ref,
                     gamma_ref, beta_ref, out_ref):
    facts = facts_ref[...]
    flt = flt_ref[...]

    iota_d = jax.lax.broadcasted_iota(jnp.int32, (D, R), 0)
    maskT = jnp.zeros((D, R), jnp.float32)
    work = flt
    for _ in range(K_FACTS):
        m = jnp.max(work, axis=0, keepdims=True)
        eq = work == m
        sel = jnp.min(jnp.where(eq, iota_d, D), axis=0, keepdims=True)
        hit = iota_d == sel
        maskT = maskT + hit.astype(jnp.float32)
        work = jnp.where(hit, -jnp.inf, work)

    aggT = aggT_ref[...]
    am = jnp.max(aggT, axis=0, keepdims=True)
    ae = jnp.exp(aggT - am)
    aw = ae / jnp.sum(ae, axis=0, keepdims=True)
    w_and, w_or, w_kofn = aw[0:1, :], aw[1:2, :], aw[2:3, :]

    S = jnp.dot(facts, maskT, preferred_element_type=jnp.float32,
                precision=jax.lax.Precision.HIGHEST)
    Q = jnp.dot(facts * facts, maskT, preferred_element_type=jnp.float32,
                precision=jax.lax.Precision.HIGHEST)
    prod = (S * S - Q) * 0.5
    mixed = w_and * prod + w_or * (S - prod) + w_kofn * (S * (1.0 / (2.0 + 1e-8)))
    act = mixed * jax.nn.sigmoid(rs_ref[...])

    iota_r = jax.lax.broadcasted_iota(jnp.int32, (B, R), 1)
    vals = act
    gated = jnp.zeros((B, R), jnp.float32)
    for _ in range(K_RULES):
        m = jnp.max(vals, axis=1, keepdims=True)
        eq = vals == m
        sel = jnp.min(jnp.where(eq, iota_r, R), axis=1, keepdims=True)
        hit = iota_r == sel
        gated = jnp.where(hit, act, gated)
        vals = jnp.where(hit, -jnp.inf, vals)

    pre = jnp.dot(facts, projWT_ref[...], preferred_element_type=jnp.float32,
                  precision=jax.lax.Precision.HIGHEST) + gated
    mu = jnp.mean(pre, axis=1, keepdims=True)
    cen = pre - mu
    var = jnp.mean(cen * cen, axis=1, keepdims=True)
    out_ref[...] = cen * jax.lax.rsqrt(var + 1e-5) * gamma_ref[...] + beta_ref[...]


def kernel(facts, fact_logits, aggregator_logits, rule_strength_raw, proj_W,
           ln_gamma, ln_beta):
    flt = fact_logits.T
    aggT = aggregator_logits.T
    rs = rule_strength_raw.reshape(1, R)
    projWT = proj_W.T
    gamma = ln_gamma.reshape(1, R)
    beta = ln_beta.reshape(1, R)
    return pl.pallas_call(
        _rule_layer_body,
        out_shape=jax.ShapeDtypeStruct((B, R), jnp.float32),
    )(facts, flt, aggT, rs, projWT, gamma, beta)


# speedup vs baseline: 23.5410x; 1.1094x over previous
import jax
import jax.numpy as jnp
from jax.experimental import pallas as pl

B, D, R = 1024, 128, 256
K_FACTS, K_RULES = 2, 8


def _rule_layer_body(facts_ref, flt_ref, aggT_ref, rs_ref, projWT_ref,
                     gamma_ref, beta_ref, out_ref):
    facts = facts_ref[...]
    flt = flt_ref[...]

    iota_d = jax.lax.broadcasted_iota(jnp.int32, (D, R), 0)
    maskT = jnp.zeros((D, R), jnp.float32)
    work = flt
    for _ in range(K_FACTS):
        m = jnp.max(work, axis=0, keepdims=True)
        eq = work == m
        sel = jnp.min(jnp.where(eq, iota_d, D), axis=0, keepdims=True)
        hit = iota_d == sel
        maskT = maskT + hit.astype(jnp.float32)
        work = jnp.where(hit, -jnp.inf, work)

    aggT = aggT_ref[...]
    am = jnp.max(aggT, axis=0, keepdims=True)
    ae = jnp.exp(aggT - am)
    aw = ae / jnp.sum(ae, axis=0, keepdims=True)
    w_and, w_or, w_kofn = aw[0:1, :], aw[1:2, :], aw[2:3, :]

    S = jnp.dot(facts, maskT, preferred_element_type=jnp.float32,
                precision=jax.lax.Precision.HIGHEST)
    Q = jnp.dot(facts * facts, maskT, preferred_element_type=jnp.float32,
                precision=jax.lax.Precision.HIGHEST)
    prod = (S * S - Q) * 0.5
    mixed = w_and * prod + w_or * (S - prod) + w_kofn * (S * (1.0 / (2.0 + 1e-8)))
    act = mixed * jax.nn.sigmoid(rs_ref[...])

    iota_r = jax.lax.broadcasted_iota(jnp.int32, (B, R), 1)
    keys = (jax.lax.bitcast_convert_type(act, jnp.int32) & ~0xFF) | (255 - iota_r)
    vals = keys
    m = jnp.zeros((B, 1), jnp.int32)
    for _ in range(K_RULES):
        m = jnp.max(vals, axis=1, keepdims=True)
        vals = jnp.where(vals == m, jnp.iinfo(jnp.int32).min, vals)
    gated = jnp.where(keys >= m, act, 0.0)

    pre = jnp.dot(facts, projWT_ref[...], preferred_element_type=jnp.float32,
                  precision=jax.lax.Precision.HIGHEST) + gated
    mu = jnp.mean(pre, axis=1, keepdims=True)
    cen = pre - mu
    var = jnp.mean(cen * cen, axis=1, keepdims=True)
    out_ref[...] = cen * jax.lax.rsqrt(var + 1e-5) * gamma_ref[...] + beta_ref[...]


def kernel(facts, fact_logits, aggregator_logits, rule_strength_raw, proj_W,
           ln_gamma, ln_beta):
    flt = fact_logits.T
    aggT = aggregator_logits.T
    rs = rule_strength_raw.reshape(1, R)
    projWT = proj_W.T
    gamma = ln_gamma.reshape(1, R)
    beta = ln_beta.reshape(1, R)
    return pl.pallas_call(
        _rule_layer_body,
        out_shape=jax.ShapeDtypeStruct((B, R), jnp.float32),
    )(facts, flt, aggT, rs, projWT, gamma, beta)


# speedup vs baseline: 29.5116x; 1.2536x over previous
import jax
import jax.numpy as jnp
from jax.experimental import pallas as pl

B, D, R = 1024, 128, 256
K_FACTS, K_RULES = 2, 8
_NT = (((1,), (1,)), ((), ()))


def _rule_layer_body(facts_ref, fl_ref, aggT_ref, rs_ref, projW_ref,
                     gamma_ref, beta_ref, out_ref):
    facts = facts_ref[...]
    fl = fl_ref[...]

    iota_d = jax.lax.broadcasted_iota(jnp.int32, (R, D), 1)
    mask = jnp.zeros((R, D), jnp.float32)
    work = fl
    for _ in range(K_FACTS):
        m = jnp.max(work, axis=1, keepdims=True)
        eq = work == m
        sel = jnp.min(jnp.where(eq, iota_d, D), axis=1, keepdims=True)
        hit = iota_d == sel
        mask = mask + hit.astype(jnp.float32)
        work = jnp.where(hit, -jnp.inf, work)

    aggT = aggT_ref[...]
    am = jnp.max(aggT, axis=0, keepdims=True)
    ae = jnp.exp(aggT - am)
    aw = ae / jnp.sum(ae, axis=0, keepdims=True)
    rstr = jax.nn.sigmoid(rs_ref[...])
    alpha = (aw[0:1, :] - aw[1:2, :]) * 0.5 * rstr
    beta = (aw[1:2, :] + aw[2:3, :] * (1.0 / (2.0 + 1e-8))) * rstr

    lhs = jnp.concatenate([facts, facts * facts], axis=0)
    SQ = jax.lax.dot_general(lhs, mask, _NT,
                             preferred_element_type=jnp.float32,
                             precision=jax.lax.Precision.HIGHEST)
    S, Q = SQ[:B, :], SQ[B:, :]
    act = alpha * (S * S - Q) + beta * S

    iota_r = jax.lax.broadcasted_iota(jnp.int32, (B, R), 1)
    keys = (jax.lax.bitcast_convert_type(act, jnp.int32) & ~0xFF) | (255 - iota_r)
    vals = keys
    m = jnp.zeros((B, 1), jnp.int32)
    for _ in range(K_RULES):
        m = jnp.max(vals, axis=1, keepdims=True)
        vals = jnp.where(vals == m, jnp.iinfo(jnp.int32).min, vals)
    gated = jnp.where(keys >= m, act, 0.0)

    proj = jax.lax.dot_general(facts, projW_ref[...], _NT,
                               preferred_element_type=jnp.float32,
                               precision=jax.lax.Precision.HIGHEST)
    pre = proj + gated
    mu = jnp.mean(pre, axis=1, keepdims=True)
    cen = pre - mu
    var = jnp.mean(cen * cen, axis=1, keepdims=True)
    out_ref[...] = cen * jax.lax.rsqrt(var + 1e-5) * gamma_ref[...] + beta_ref[...]


def kernel(facts, fact_logits, aggregator_logits, rule_strength_raw, proj_W,
           ln_gamma, ln_beta):
    aggT = aggregator_logits.T
    rs = rule_strength_raw.reshape(1, R)
    gamma = ln_gamma.reshape(1, R)
    beta = ln_beta.reshape(1, R)
    return pl.pallas_call(
        _rule_layer_body,
        out_shape=jax.ShapeDtypeStruct((B, R), jnp.float32),
    )(facts, fact_logits, aggT, rs, proj_W, gamma, beta)
